# baseline (speedup 1.0000x reference)
import functools

import jax
import jax.numpy as jnp
from jax import lax
from jax.experimental import pallas as pl
from jax.experimental.pallas import tpu as pltpu
from jax.experimental.pallas import tpu_sc as plsc

N = 10000
D = 128
E = 320000

NW = 32
ROWS_W = 320
N_PAD = NW * ROWS_W
CHUNK = 2000
GB = 128
TCB = 2000


def _tables_body(x_ref, pos_ref, wh1_ref, bh1_ref, wh2_ref, bh2_ref,
                 wf3_ref, wfx_ref, bf_ref, s_ref, t_ref):
    x = x_ref[...]
    pos = pos_ref[...]
    h = jnp.maximum(
        jnp.dot(x, wh1_ref[...], preferred_element_type=jnp.float32)
        + bh1_ref[...], 0.0)
    delta = jnp.dot(h, wh2_ref[...],
                    preferred_element_type=jnp.float32) + bh2_ref[...]
    s_ref[...] = (jnp.dot(pos, wf3_ref[...], preferred_element_type=jnp.float32)
                  + jnp.dot(x, wfx_ref[...], preferred_element_type=jnp.float32))
    t_ref[...] = (jnp.dot(delta - pos, wf3_ref[...],
                          preferred_element_type=jnp.float32) + bf_ref[...])


def _out_body(x_ref, t_ref, m_ref, wg_ref, bg_ref, o_ref):
    a = t_ref[...] + m_ref[...]
    agg = jnp.where(jnp.isfinite(a), a, 0.0)
    o_ref[...] = (x_ref[...]
                  + jnp.dot(agg, wg_ref[...], preferred_element_type=jnp.float32)
                  + bg_ref[...])


def _segmax_body(s_hbm, src_hbm, dst_hbm, out_hbm,
                 acc_v, dstc_v, srcc_v, cdst_v, csrc_v, rows_v, sem):
    c = lax.axis_index("c")
    s = lax.axis_index("s")
    wid = s * 2 + c
    lo = wid * ROWS_W
    lanes = lax.iota(jnp.int32, 16)
    neg = jnp.full((16,), -jnp.inf, jnp.float32)

    def init_row(i, carry):
        for j in range(D // 16):
            acc_v[i, pl.ds(16 * j, 16)] = neg
        return carry
    lax.fori_loop(0, ROWS_W, init_row, 0)

    def per_chunk(ci, carry):
        base_e = ci * CHUNK
        pltpu.sync_copy(dst_hbm.at[pl.ds(base_e, CHUNK)], dstc_v)
        pltpu.sync_copy(src_hbm.at[pl.ds(base_e, CHUNK)], srcc_v)

        def per_group(g, cnt):
            dv = dstc_v[pl.ds(g * 16, 16)]
            sv = srcc_v[pl.ds(g * 16, 16)]
            dl = dv - lo
            msk = (dl >= 0) & (dl < ROWS_W)
            pref = plsc.cumsum(msk.astype(jnp.int32))
            pos_ix = cnt + pref - 1
            plsc.store_scatter(cdst_v, [pos_ix], dl, mask=msk)
            plsc.store_scatter(csrc_v, [pos_ix], sv, mask=msk)
            return cnt + jnp.max(pref)
        cnt = lax.fori_loop(0, CHUNK // 16, per_group, jnp.int32(0))

        zeros16 = jnp.zeros((16,), jnp.int32)
        for j in range(GB // 16):
            csrc_v[pl.ds(cnt + 16 * j, 16)] = zeros16

        def per_batch(b, carry2):
            bb = b * GB
            pltpu.async_copy(s_hbm.at[csrc_v.at[pl.ds(bb, GB)]],
                             rows_v, sem).wait()

            def per_egroup(g, carry3):
                eg = bb + g * 16
                dvec = cdst_v[pl.ds(eg, 16)]
                for l in range(16):
                    @pl.when(eg + l < cnt)
                    def _():
                        r = jnp.max(jnp.where(lanes == l, dvec, 0))
                        for j in range(D // 16):
                            sl = pl.ds(16 * j, 16)
                            acc_v[r, sl] = jnp.maximum(acc_v[r, sl],
                                                       rows_v[g * 16 + l, sl])
                return carry3
            lax.fori_loop(0, GB // 16, per_egroup, 0)
            return carry2
        nb = (cnt + GB - 1) // GB
        lax.fori_loop(0, nb, per_batch, 0)
        return carry
    lax.fori_loop(0, E // CHUNK, per_chunk, 0)

    pltpu.sync_copy(acc_v, out_hbm.at[pl.ds(lo, ROWS_W)])


_segmax = functools.partial(
    pl.kernel,
    mesh=plsc.VectorSubcoreMesh(core_axis_name="c", subcore_axis_name="s"),
    out_type=jax.ShapeDtypeStruct((N_PAD, D), jnp.float32),
    scratch_types=[
        pltpu.VMEM((ROWS_W, D), jnp.float32),
        pltpu.VMEM((CHUNK,), jnp.int32),
        pltpu.VMEM((CHUNK,), jnp.int32),
        pltpu.VMEM((CHUNK + 16,), jnp.int32),
        pltpu.VMEM((CHUNK + GB + 16,), jnp.int32),
        pltpu.VMEM((GB, D), jnp.float32),
        pltpu.SemaphoreType.DMA,
    ],
    compiler_params=pltpu.CompilerParams(needs_layout_passes=False),
)(_segmax_body)


def kernel(x, pos, edge_index, Wh1, bh1, Wh2, bh2, Wf, bf, Wg, bg):
    src = edge_index[0]
    dst = edge_index[1]
    Wf3 = Wf[:3]
    Wfx = Wf[3:]

    grid = (N + TCB - 1) // TCB
    row_spec = pl.BlockSpec((TCB, D), lambda i: (i, 0))
    pos_spec = pl.BlockSpec((TCB, 3), lambda i: (i, 0))
    full = lambda shape: pl.BlockSpec(shape, lambda i: (0,) * len(shape))

    S, T = pl.pallas_call(
        _tables_body,
        grid=(grid,),
        in_specs=[
            row_spec, pos_spec,
            full(Wh1.shape), full((1, 64)), full(Wh2.shape), full((1, 3)),
            full(Wf3.shape), full(Wfx.shape), full((1, D)),
        ],
        out_specs=[row_spec, row_spec],
        out_shape=[jax.ShapeDtypeStruct((N, D), jnp.float32),
                   jax.ShapeDtypeStruct((N, D), jnp.float32)],
    )(x, pos, Wh1, bh1.reshape(1, 64), Wh2, bh2.reshape(1, 3),
      Wf3, Wfx, bf.reshape(1, D))

    M = _segmax(S, src, dst)[:N]

    out = pl.pallas_call(
        _out_body,
        grid=(grid,),
        in_specs=[row_spec, row_spec, row_spec, full(Wg.shape), full((1, D))],
        out_specs=row_spec,
        out_shape=jax.ShapeDtypeStruct((N, D), jnp.float32),
    )(x, T, M, Wg, bg.reshape(1, D))
    return out


# speedup vs baseline: 4.0934x; 4.0934x over previous
import functools

import jax
import jax.numpy as jnp
from jax import lax
from jax.experimental import pallas as pl
from jax.experimental.pallas import tpu as pltpu
from jax.experimental.pallas import tpu_sc as plsc

N = 10000
D = 128
E = 320000

NW = 32
ROWS_W = 320
N_PAD = NW * ROWS_W
CHUNK = 8000
GB = 128
TCB = 2000


def _tables_body(x_ref, pos_ref, wh1_ref, bh1_ref, wh2_ref, bh2_ref,
                 wf3_ref, wfx_ref, bf_ref, s_ref, t_ref):
    x = x_ref[...]
    pos = pos_ref[...]
    h = jnp.maximum(
        jnp.dot(x, wh1_ref[...], preferred_element_type=jnp.float32)
        + bh1_ref[...], 0.0)
    delta = jnp.dot(h, wh2_ref[...],
                    preferred_element_type=jnp.float32) + bh2_ref[...]
    s_ref[...] = (jnp.dot(pos, wf3_ref[...], preferred_element_type=jnp.float32)
                  + jnp.dot(x, wfx_ref[...], preferred_element_type=jnp.float32))
    t_ref[...] = (jnp.dot(delta - pos, wf3_ref[...],
                          preferred_element_type=jnp.float32) + bf_ref[...])


def _out_body(x_ref, t_ref, m_ref, wg_ref, bg_ref, o_ref):
    a = t_ref[...] + m_ref[...]
    agg = jnp.where(jnp.isfinite(a), a, 0.0)
    o_ref[...] = (x_ref[...]
                  + jnp.dot(agg, wg_ref[...], preferred_element_type=jnp.float32)
                  + bg_ref[...])


def _segmax_body(s_hbm, src_hbm, dst_hbm, out_hbm,
                 acc_v, dstc_v, srcc_v, cdst_v, csrc_v, rows_v, sem):
    c = lax.axis_index("c")
    s = lax.axis_index("s")
    wid = s * 2 + c
    lo = wid * ROWS_W
    lanes = lax.iota(jnp.int32, 16)
    neg = jnp.full((16,), -jnp.inf, jnp.float32)

    def init_row(i, carry):
        for j in range(D // 16):
            acc_v[i, pl.ds(16 * j, 16)] = neg
        return carry
    lax.fori_loop(0, ROWS_W + 1, init_row, 0)

    def per_chunk(ci, carry):
        base_e = ci * CHUNK
        pltpu.sync_copy(dst_hbm.at[pl.ds(base_e, CHUNK)], dstc_v)
        pltpu.sync_copy(src_hbm.at[pl.ds(base_e, CHUNK)], srcc_v)

        def per_group(g, cnt_vec):
            dv = dstc_v[pl.ds(g * 16, 16)]
            sv = srcc_v[pl.ds(g * 16, 16)]
            dl = dv - lo
            msk = (dl >= 0) & (dl < ROWS_W)
            pref = plsc.cumsum(msk.astype(jnp.int32))
            pos_ix = cnt_vec + pref - 1
            plsc.store_scatter(cdst_v, [pos_ix], dl, mask=msk)
            plsc.store_scatter(csrc_v, [pos_ix], sv, mask=msk)
            return cnt_vec + plsc.all_reduce_population_count(msk)
        cnt_vec = lax.fori_loop(0, CHUNK // 16, per_group,
                                jnp.zeros((16,), jnp.int32))
        cnt = jnp.max(cnt_vec)

        zeros16 = jnp.zeros((16,), jnp.int32)
        sent16 = jnp.full((16,), ROWS_W, jnp.int32)
        for j in range(GB // 16):
            tail_ix = cnt_vec + (lanes + 16 * j)
            plsc.store_scatter(csrc_v, [tail_ix], zeros16)
            plsc.store_scatter(cdst_v, [tail_ix], sent16)

        def per_batch(b, carry2):
            bb = b * GB
            pltpu.async_copy(s_hbm.at[csrc_v.at[pl.ds(bb, GB)]],
                             rows_v, sem).wait()
            rem = cnt - bb
            ng = (jnp.minimum(rem, GB) + 15) // 16

            def per_egroup(g, carry3):
                eg = bb + g * 16
                dvec = cdst_v[pl.ds(eg, 16)]
                for l in range(16):
                    rvec = dvec.at[jnp.full((16,), l, jnp.int32)].get(
                        mode="promise_in_bounds")
                    p = g * 16 + l
                    for j in range(D // 16):
                        col = lanes + (16 * j)
                        cur = plsc.load_gather(acc_v, [rvec, col])
                        new = jnp.maximum(cur, rows_v[p, pl.ds(16 * j, 16)])
                        plsc.store_scatter(acc_v, [rvec, col], new)
                return carry3
            lax.fori_loop(0, ng, per_egroup, 0)
            return carry2
        nb = (cnt + GB - 1) // GB
        lax.fori_loop(0, nb, per_batch, 0)
        return carry
    lax.fori_loop(0, E // CHUNK, per_chunk, 0)

    pltpu.sync_copy(acc_v.at[pl.ds(0, ROWS_W)], out_hbm.at[pl.ds(lo, ROWS_W)])


_segmax = functools.partial(
    pl.kernel,
    mesh=plsc.VectorSubcoreMesh(core_axis_name="c", subcore_axis_name="s"),
    out_type=jax.ShapeDtypeStruct((N_PAD, D), jnp.float32),
    scratch_types=[
        pltpu.VMEM((ROWS_W + 1, D), jnp.float32),
        pltpu.VMEM((CHUNK,), jnp.int32),
        pltpu.VMEM((CHUNK,), jnp.int32),
        pltpu.VMEM((CHUNK + GB + 16,), jnp.int32),
        pltpu.VMEM((CHUNK + GB + 16,), jnp.int32),
        pltpu.VMEM((GB, D), jnp.float32),
        pltpu.SemaphoreType.DMA,
    ],
    compiler_params=pltpu.CompilerParams(needs_layout_passes=False),
)(_segmax_body)


def kernel(x, pos, edge_index, Wh1, bh1, Wh2, bh2, Wf, bf, Wg, bg):
    src = edge_index[0]
    dst = edge_index[1]
    Wf3 = Wf[:3]
    Wfx = Wf[3:]

    grid = (N + TCB - 1) // TCB
    row_spec = pl.BlockSpec((TCB, D), lambda i: (i, 0))
    pos_spec = pl.BlockSpec((TCB, 3), lambda i: (i, 0))
    full = lambda shape: pl.BlockSpec(shape, lambda i: (0,) * len(shape))

    S, T = pl.pallas_call(
        _tables_body,
        grid=(grid,),
        in_specs=[
            row_spec, pos_spec,
            full(Wh1.shape), full((1, 64)), full(Wh2.shape), full((1, 3)),
            full(Wf3.shape), full(Wfx.shape), full((1, D)),
        ],
        out_specs=[row_spec, row_spec],
        out_shape=[jax.ShapeDtypeStruct((N, D), jnp.float32),
                   jax.ShapeDtypeStruct((N, D), jnp.float32)],
    )(x, pos, Wh1, bh1.reshape(1, 64), Wh2, bh2.reshape(1, 3),
      Wf3, Wfx, bf.reshape(1, D))

    M = _segmax(S, src, dst)[:N]

    out = pl.pallas_call(
        _out_body,
        grid=(grid,),
        in_specs=[row_spec, row_spec, row_spec, full(Wg.shape), full((1, D))],
        out_specs=row_spec,
        out_shape=jax.ShapeDtypeStruct((N, D), jnp.float32),
    )(x, T, M, Wg, bg.reshape(1, D))
    return out


# speedup vs baseline: 4.0970x; 1.0009x over previous
import functools

import jax
import jax.numpy as jnp
from jax import lax
from jax.experimental import pallas as pl
from jax.experimental.pallas import tpu as pltpu
from jax.experimental.pallas import tpu_sc as plsc

N = 10000
D = 128
E = 320000

NW = 32
ROWS_W = 320
N_PAD = NW * ROWS_W
CHUNK = 8000
GB = 128
TCB = 2000


def _tables_body(x_ref, pos_ref, wh1_ref, bh1_ref, wh2_ref, bh2_ref,
                 wf3_ref, wfx_ref, bf_ref, s_ref, t_ref):
    x = x_ref[...]
    pos = pos_ref[...]
    h = jnp.maximum(
        jnp.dot(x, wh1_ref[...], preferred_element_type=jnp.float32)
        + bh1_ref[...], 0.0)
    delta = jnp.dot(h, wh2_ref[...],
                    preferred_element_type=jnp.float32) + bh2_ref[...]
    s_ref[...] = (jnp.dot(pos, wf3_ref[...], preferred_element_type=jnp.float32)
                  + jnp.dot(x, wfx_ref[...], preferred_element_type=jnp.float32))
    t_ref[...] = (jnp.dot(delta - pos, wf3_ref[...],
                          preferred_element_type=jnp.float32) + bf_ref[...])


def _out_body(x_ref, t_ref, m_ref, wg_ref, bg_ref, o_ref):
    a = t_ref[...] + m_ref[...]
    agg = jnp.where(jnp.isfinite(a), a, 0.0)
    o_ref[...] = (x_ref[...]
                  + jnp.dot(agg, wg_ref[...], preferred_element_type=jnp.float32)
                  + bg_ref[...])


def _segmax_body(s_hbm, src_hbm, dst_hbm, out_hbm,
                 acc_v, dstc_v, srcc_v, cdst_v, csrc_v, rows_v,
                 sem_d, sem_s, sem_g0, sem_g1):
    c = lax.axis_index("c")
    s = lax.axis_index("s")
    wid = s * 2 + c
    lo = wid * ROWS_W
    lanes = lax.iota(jnp.int32, 16)
    neg = jnp.full((16,), -jnp.inf, jnp.float32)

    pltpu.make_async_copy(dst_hbm.at[pl.ds(0, CHUNK)], dstc_v, sem_d).start()
    pltpu.make_async_copy(src_hbm.at[pl.ds(0, CHUNK)], srcc_v, sem_s).start()

    def init_row(i, carry):
        for j in range(D // 16):
            acc_v[i, pl.ds(16 * j, 16)] = neg
        return carry
    lax.fori_loop(0, ROWS_W + 1, init_row, 0)

    def per_chunk(ci, carry):
        base_e = ci * CHUNK
        pltpu.make_async_copy(dst_hbm.at[pl.ds(base_e, CHUNK)],
                              dstc_v, sem_d).wait()
        pltpu.make_async_copy(src_hbm.at[pl.ds(base_e, CHUNK)],
                              srcc_v, sem_s).wait()

        def per_group(g, cnt_vec):
            dv = dstc_v[pl.ds(g * 16, 16)]
            sv = srcc_v[pl.ds(g * 16, 16)]
            dl = dv - lo
            msk = (dl >= 0) & (dl < ROWS_W)
            pref = plsc.cumsum(msk.astype(jnp.int32))
            pos_ix = cnt_vec + pref - 1
            plsc.store_scatter(cdst_v, [pos_ix], dl, mask=msk)
            plsc.store_scatter(csrc_v, [pos_ix], sv, mask=msk)
            return cnt_vec + plsc.all_reduce_population_count(msk)
        cnt_vec = lax.fori_loop(0, CHUNK // 16, per_group,
                                jnp.zeros((16,), jnp.int32))
        cnt = jnp.max(cnt_vec)

        zeros16 = jnp.zeros((16,), jnp.int32)
        sent16 = jnp.full((16,), ROWS_W, jnp.int32)
        for j in range(GB // 16):
            tail_ix = cnt_vec + (lanes + 16 * j)
            plsc.store_scatter(csrc_v, [tail_ix], zeros16)
            plsc.store_scatter(cdst_v, [tail_ix], sent16)

        @pl.when(ci + 1 < E // CHUNK)
        def _():
            nxt = base_e + CHUNK
            pltpu.make_async_copy(dst_hbm.at[pl.ds(nxt, CHUNK)],
                                  dstc_v, sem_d).start()
            pltpu.make_async_copy(src_hbm.at[pl.ds(nxt, CHUNK)],
                                  srcc_v, sem_s).start()

        nb = (cnt + GB - 1) // GB
        sems = (sem_g0, sem_g1)

        @pl.when(nb > 0)
        def _():
            pltpu.make_async_copy(s_hbm.at[csrc_v.at[pl.ds(0, GB)]],
                                  rows_v.at[pl.ds(0, GB)], sem_g0).start()

        @pl.when(nb > 1)
        def _():
            pltpu.make_async_copy(s_hbm.at[csrc_v.at[pl.ds(GB, GB)]],
                                  rows_v.at[pl.ds(GB, GB)], sem_g1).start()

        def per_pair(p, carry2):
            for buf in range(2):
                b = 2 * p + buf

                @pl.when(b < nb)
                def _(b=b, buf=buf):
                    bb = b * GB
                    pltpu.make_async_copy(
                        s_hbm.at[csrc_v.at[pl.ds(bb, GB)]],
                        rows_v.at[pl.ds(buf * GB, GB)], sems[buf]).wait()
                    rem = cnt - bb
                    ng = (jnp.minimum(rem, GB) + 15) // 16

                    def per_egroup(g, carry3):
                        eg = bb + g * 16
                        dvec = cdst_v[pl.ds(eg, 16)]
                        for l in range(16):
                            rvec = dvec.at[jnp.full((16,), l, jnp.int32)].get(
                                mode="promise_in_bounds")
                            p_row = buf * GB + g * 16 + l
                            for j in range(D // 16):
                                col = lanes + (16 * j)
                                cur = plsc.load_gather(acc_v, [rvec, col])
                                new = jnp.maximum(
                                    cur, rows_v[p_row, pl.ds(16 * j, 16)])
                                plsc.store_scatter(acc_v, [rvec, col], new)
                        return carry3
                    lax.fori_loop(0, ng, per_egroup, 0)

                    @pl.when(b + 2 < nb)
                    def _():
                        pltpu.make_async_copy(
                            s_hbm.at[csrc_v.at[pl.ds(bb + 2 * GB, GB)]],
                            rows_v.at[pl.ds(buf * GB, GB)], sems[buf]).start()
            return carry2
        lax.fori_loop(0, (nb + 1) // 2, per_pair, 0)
        return carry
    lax.fori_loop(0, E // CHUNK, per_chunk, 0)

    pltpu.sync_copy(acc_v.at[pl.ds(0, ROWS_W)], out_hbm.at[pl.ds(lo, ROWS_W)])


_segmax = functools.partial(
    pl.kernel,
    mesh=plsc.VectorSubcoreMesh(core_axis_name="c", subcore_axis_name="s"),
    out_type=jax.ShapeDtypeStruct((N_PAD, D), jnp.float32),
    scratch_types=[
        pltpu.VMEM((ROWS_W + 1, D), jnp.float32),
        pltpu.VMEM((CHUNK,), jnp.int32),
        pltpu.VMEM((CHUNK,), jnp.int32),
        pltpu.VMEM((CHUNK + GB + 16,), jnp.int32),
        pltpu.VMEM((CHUNK + GB + 16,), jnp.int32),
        pltpu.VMEM((2 * GB, D), jnp.float32),
        pltpu.SemaphoreType.DMA,
        pltpu.SemaphoreType.DMA,
        pltpu.SemaphoreType.DMA,
        pltpu.SemaphoreType.DMA,
    ],
    compiler_params=pltpu.CompilerParams(needs_layout_passes=False),
)(_segmax_body)


def kernel(x, pos, edge_index, Wh1, bh1, Wh2, bh2, Wf, bf, Wg, bg):
    src = edge_index[0]
    dst = edge_index[1]
    Wf3 = Wf[:3]
    Wfx = Wf[3:]

    grid = (N + TCB - 1) // TCB
    row_spec = pl.BlockSpec((TCB, D), lambda i: (i, 0))
    pos_spec = pl.BlockSpec((TCB, 3), lambda i: (i, 0))
    full = lambda shape: pl.BlockSpec(shape, lambda i: (0,) * len(shape))

    S, T = pl.pallas_call(
        _tables_body,
        grid=(grid,),
        in_specs=[
            row_spec, pos_spec,
            full(Wh1.shape), full((1, 64)), full(Wh2.shape), full((1, 3)),
            full(Wf3.shape), full(Wfx.shape), full((1, D)),
        ],
        out_specs=[row_spec, row_spec],
        out_shape=[jax.ShapeDtypeStruct((N, D), jnp.float32),
                   jax.ShapeDtypeStruct((N, D), jnp.float32)],
    )(x, pos, Wh1, bh1.reshape(1, 64), Wh2, bh2.reshape(1, 3),
      Wf3, Wfx, bf.reshape(1, D))

    M = _segmax(S, src, dst)[:N]

    out = pl.pallas_call(
        _out_body,
        grid=(grid,),
        in_specs=[row_spec, row_spec, row_spec, full(Wg.shape), full((1, D))],
        out_specs=row_spec,
        out_shape=jax.ShapeDtypeStruct((N, D), jnp.float32),
    )(x, T, M, Wg, bg.reshape(1, D))
    return out


# speedup vs baseline: 32.2404x; 7.8693x over previous
import functools

import jax
import jax.numpy as jnp
from jax import lax
from jax.experimental import pallas as pl
from jax.experimental.pallas import tpu as pltpu
from jax.experimental.pallas import tpu_sc as plsc

N = 10000
D = 128
E = 320000

NW = 32
ROWS_W = 320
N_PAD = NW * ROWS_W
CHUNK = 8000
CAP = 19800
GB = 128
TCB = 2000


def _tables_body(x_ref, pos_ref, wh1_ref, bh1_ref, wh2_ref, bh2_ref,
                 wf3_ref, wfx_ref, bf_ref, s_ref, t_ref):
    x = x_ref[...]
    pos = pos_ref[...]
    h = jnp.maximum(
        jnp.dot(x, wh1_ref[...], preferred_element_type=jnp.float32)
        + bh1_ref[...], 0.0)
    delta = jnp.dot(h, wh2_ref[...],
                    preferred_element_type=jnp.float32) + bh2_ref[...]
    s_ref[...] = (jnp.dot(pos, wf3_ref[...], preferred_element_type=jnp.float32)
                  + jnp.dot(x, wfx_ref[...], preferred_element_type=jnp.float32))
    t_ref[...] = (jnp.dot(delta - pos, wf3_ref[...],
                          preferred_element_type=jnp.float32) + bf_ref[...])


def _out_body(x_ref, t_ref, m_ref, wg_ref, bg_ref, o_ref):
    a = t_ref[...] + m_ref[...]
    agg = jnp.where(jnp.isfinite(a), a, 0.0)
    o_ref[...] = (x_ref[...]
                  + jnp.dot(agg, wg_ref[...], preferred_element_type=jnp.float32)
                  + bg_ref[...])


def _segmax_body(s_hbm, src_hbm, dst_hbm, out_hbm,
                 acc_v, dstc_v, srcc_v, cdst_v, csrc_v, rows_v,
                 sem_d, sem_s, sem_g0, sem_g1):
    c = lax.axis_index("c")
    s = lax.axis_index("s")
    wid = s * 2 + c
    lo = wid * ROWS_W
    lanes = lax.iota(jnp.int32, 16)
    neg = jnp.full((16,), -jnp.inf, jnp.float32)
    zeros16 = jnp.zeros((16,), jnp.int32)
    sent16 = jnp.full((16,), ROWS_W, jnp.int32)

    pltpu.make_async_copy(dst_hbm.at[pl.ds(0, CHUNK)], dstc_v, sem_d).start()
    pltpu.make_async_copy(src_hbm.at[pl.ds(0, CHUNK)], srcc_v, sem_s).start()

    def init_row(i, carry):
        for j in range(D // 16):
            acc_v[i, pl.ds(16 * j, 16)] = neg
        return carry
    lax.fori_loop(0, ROWS_W + 1, init_row, 0)

    def accumulate(cnt_vec):
        cnt = jnp.max(cnt_vec)
        for j in range(GB // 16):
            tail_ix = cnt_vec + (lanes + 16 * j)
            plsc.store_scatter(csrc_v, [tail_ix], zeros16)
            plsc.store_scatter(cdst_v, [tail_ix], sent16)

        nb = (cnt + GB - 1) // GB
        sems = (sem_g0, sem_g1)

        @pl.when(nb > 0)
        def _():
            pltpu.make_async_copy(s_hbm.at[csrc_v.at[pl.ds(0, GB)]],
                                  rows_v.at[pl.ds(0, GB)], sem_g0).start()

        @pl.when(nb > 1)
        def _():
            pltpu.make_async_copy(s_hbm.at[csrc_v.at[pl.ds(GB, GB)]],
                                  rows_v.at[pl.ds(GB, GB)], sem_g1).start()

        def per_pair(p, carry2):
            for buf in range(2):
                b = 2 * p + buf

                @pl.when(b < nb)
                def _(b=b, buf=buf):
                    bb = b * GB
                    pltpu.make_async_copy(
                        s_hbm.at[csrc_v.at[pl.ds(bb, GB)]],
                        rows_v.at[pl.ds(buf * GB, GB)], sems[buf]).wait()
                    rem = cnt - bb
                    ng = (jnp.minimum(rem, GB) + 15) // 16

                    def per_egroup(g, carry3):
                        eg = bb + g * 16
                        dvec = cdst_v[pl.ds(eg, 16)]
                        for l in range(16):
                            rvec = dvec.at[jnp.full((16,), l, jnp.int32)].get(
                                mode="promise_in_bounds")
                            p_row = buf * GB + g * 16 + l
                            curs = [plsc.load_gather(
                                        acc_v, [rvec, lanes + 16 * j])
                                    for j in range(D // 16)]
                            news = [jnp.maximum(
                                        curs[j],
                                        rows_v[p_row, pl.ds(16 * j, 16)])
                                    for j in range(D // 16)]
                            for j in range(D // 16):
                                plsc.store_scatter(
                                    acc_v, [rvec, lanes + 16 * j], news[j])
                        return carry3
                    lax.fori_loop(0, ng, per_egroup, 0)

                    @pl.when(b + 2 < nb)
                    def _():
                        pltpu.make_async_copy(
                            s_hbm.at[csrc_v.at[pl.ds(bb + 2 * GB, GB)]],
                            rows_v.at[pl.ds(buf * GB, GB)], sems[buf]).start()
            return carry2
        lax.fori_loop(0, (nb + 1) // 2, per_pair, 0)

    def per_chunk(ci, cnt_vec):
        base_e = ci * CHUNK
        pltpu.make_async_copy(dst_hbm.at[pl.ds(base_e, CHUNK)],
                              dstc_v, sem_d).wait()
        pltpu.make_async_copy(src_hbm.at[pl.ds(base_e, CHUNK)],
                              srcc_v, sem_s).wait()

        def per_quad(q, cnt_vec2):
            prefs, dls, svs, msks = [], [], [], []
            for u in range(4):
                g = q * 4 + u
                dv = dstc_v[pl.ds(g * 16, 16)]
                sv = srcc_v[pl.ds(g * 16, 16)]
                dl = dv - lo
                msk = (dl >= 0) & (dl < ROWS_W)
                prefs.append(plsc.cumsum(msk.astype(jnp.int32)))
                dls.append(dl)
                svs.append(sv)
                msks.append(msk)
            for u in range(4):
                pos_ix = cnt_vec2 + prefs[u] - 1
                plsc.store_scatter(cdst_v, [pos_ix], dls[u], mask=msks[u])
                plsc.store_scatter(csrc_v, [pos_ix], svs[u], mask=msks[u])
                last = prefs[u].at[jnp.full((16,), 15, jnp.int32)].get(
                    mode="promise_in_bounds")
                cnt_vec2 = cnt_vec2 + last
            return cnt_vec2
        cnt_vec = lax.fori_loop(0, CHUNK // 64, per_quad, cnt_vec)

        @pl.when(ci + 1 < E // CHUNK)
        def _():
            nxt = base_e + CHUNK
            pltpu.make_async_copy(dst_hbm.at[pl.ds(nxt, CHUNK)],
                                  dstc_v, sem_d).start()
            pltpu.make_async_copy(src_hbm.at[pl.ds(nxt, CHUNK)],
                                  srcc_v, sem_s).start()

        full = jnp.max(cnt_vec) > CAP - CHUNK

        @pl.when(full)
        def _():
            accumulate(cnt_vec)
        return jnp.where(full, jnp.zeros((16,), jnp.int32), cnt_vec)

    cnt_vec = lax.fori_loop(0, E // CHUNK, per_chunk,
                            jnp.zeros((16,), jnp.int32))
    accumulate(cnt_vec)

    pltpu.sync_copy(acc_v.at[pl.ds(0, ROWS_W)], out_hbm.at[pl.ds(lo, ROWS_W)])


_segmax = functools.partial(
    pl.kernel,
    mesh=plsc.VectorSubcoreMesh(core_axis_name="c", subcore_axis_name="s"),
    out_type=jax.ShapeDtypeStruct((N_PAD, D), jnp.float32),
    scratch_types=[
        pltpu.VMEM((ROWS_W + 1, D), jnp.float32),
        pltpu.VMEM((CHUNK,), jnp.int32),
        pltpu.VMEM((CHUNK,), jnp.int32),
        pltpu.VMEM((CAP + GB + 16,), jnp.int32),
        pltpu.VMEM((CAP + GB + 16,), jnp.int32),
        pltpu.VMEM((2 * GB, D), jnp.float32),
        pltpu.SemaphoreType.DMA,
        pltpu.SemaphoreType.DMA,
        pltpu.SemaphoreType.DMA,
        pltpu.SemaphoreType.DMA,
    ],
    compiler_params=pltpu.CompilerParams(needs_layout_passes=False),
)(_segmax_body)


def kernel(x, pos, edge_index, Wh1, bh1, Wh2, bh2, Wf, bf, Wg, bg):
    src = edge_index[0]
    dst = edge_index[1]
    Wf3 = Wf[:3]
    Wfx = Wf[3:]

    grid = (N + TCB - 1) // TCB
    row_spec = pl.BlockSpec((TCB, D), lambda i: (i, 0))
    pos_spec = pl.BlockSpec((TCB, 3), lambda i: (i, 0))
    full = lambda shape: pl.BlockSpec(shape, lambda i: (0,) * len(shape))

    S, T = pl.pallas_call(
        _tables_body,
        grid=(grid,),
        in_specs=[
            row_spec, pos_spec,
            full(Wh1.shape), full((1, 64)), full(Wh2.shape), full((1, 3)),
            full(Wf3.shape), full(Wfx.shape), full((1, D)),
        ],
        out_specs=[row_spec, row_spec],
        out_shape=[jax.ShapeDtypeStruct((N, D), jnp.float32),
                   jax.ShapeDtypeStruct((N, D), jnp.float32)],
    )(x, pos, Wh1, bh1.reshape(1, 64), Wh2, bh2.reshape(1, 3),
      Wf3, Wfx, bf.reshape(1, D))

    M = _segmax(S, src, dst)

    out = pl.pallas_call(
        _out_body,
        grid=(grid,),
        in_specs=[row_spec, row_spec, row_spec, full(Wg.shape), full((1, D))],
        out_specs=row_spec,
        out_shape=jax.ShapeDtypeStruct((N, D), jnp.float32),
    )(x, T, M, Wg, bg.reshape(1, D))
    return out


# speedup vs baseline: 34.2846x; 1.0634x over previous
import functools

import jax
import jax.numpy as jnp
from jax import lax
from jax.experimental import pallas as pl
from jax.experimental.pallas import tpu as pltpu
from jax.experimental.pallas import tpu_sc as plsc

N = 10000
D = 128
E = 320000

NSUB = 16
ROWS_W = 640
N_PAD = NSUB * ROWS_W
ECH = E // 2
CHUNK = 3200
CAP = 10000
GB = 64
TCB = 2000


def _tables_body(x_ref, pos_ref, wh1_ref, bh1_ref, wh2_ref, bh2_ref,
                 wf3_ref, wfx_ref, bf_ref, s_ref, t_ref):
    x = x_ref[...]
    pos = pos_ref[...]
    h = jnp.maximum(
        jnp.dot(x, wh1_ref[...], preferred_element_type=jnp.float32)
        + bh1_ref[...], 0.0)
    delta = jnp.dot(h, wh2_ref[...],
                    preferred_element_type=jnp.float32) + bh2_ref[...]
    s_ref[...] = (jnp.dot(pos, wf3_ref[...], preferred_element_type=jnp.float32)
                  + jnp.dot(x, wfx_ref[...], preferred_element_type=jnp.float32))
    t_ref[...] = (jnp.dot(delta - pos, wf3_ref[...],
                          preferred_element_type=jnp.float32) + bf_ref[...])


def _out_body(x_ref, t_ref, m0_ref, m1_ref, wg_ref, bg_ref, o_ref):
    a = t_ref[...] + jnp.maximum(m0_ref[...], m1_ref[...])
    agg = jnp.where(jnp.isfinite(a), a, 0.0)
    o_ref[...] = (x_ref[...]
                  + jnp.dot(agg, wg_ref[...], preferred_element_type=jnp.float32)
                  + bg_ref[...])


def _segmax_body(s_hbm, src_hbm, dst_hbm, out_hbm,
                 acc_v, dstc_v, srcc_v, cdst_v, csrc_v, rows_v,
                 sem_d, sem_s, sem_g0, sem_g1):
    c = lax.axis_index("c")
    s = lax.axis_index("s")
    lo = s * ROWS_W
    ebase = c * ECH
    lanes = lax.iota(jnp.int32, 16)
    neg = jnp.full((16,), -jnp.inf, jnp.float32)
    zeros16 = jnp.zeros((16,), jnp.int32)
    sent16 = jnp.full((16,), ROWS_W, jnp.int32)

    pltpu.make_async_copy(dst_hbm.at[pl.ds(ebase, CHUNK)],
                          dstc_v, sem_d).start()
    pltpu.make_async_copy(src_hbm.at[pl.ds(ebase, CHUNK)],
                          srcc_v, sem_s).start()

    def init_row(i, carry):
        for j in range(D // 16):
            acc_v[i, pl.ds(16 * j, 16)] = neg
        return carry
    lax.fori_loop(0, ROWS_W + 1, init_row, 0)

    def accumulate(cnt_vec):
        cnt = jnp.max(cnt_vec)
        for j in range(GB // 16):
            tail_ix = cnt_vec + (lanes + 16 * j)
            plsc.store_scatter(csrc_v, [tail_ix], zeros16)
            plsc.store_scatter(cdst_v, [tail_ix], sent16)

        nb = (cnt + GB - 1) // GB
        sems = (sem_g0, sem_g1)

        @pl.when(nb > 0)
        def _():
            pltpu.make_async_copy(s_hbm.at[csrc_v.at[pl.ds(0, GB)]],
                                  rows_v.at[pl.ds(0, GB)], sem_g0).start()

        @pl.when(nb > 1)
        def _():
            pltpu.make_async_copy(s_hbm.at[csrc_v.at[pl.ds(GB, GB)]],
                                  rows_v.at[pl.ds(GB, GB)], sem_g1).start()

        def per_pair(p, carry2):
            for buf in range(2):
                b = 2 * p + buf

                @pl.when(b < nb)
                def _(b=b, buf=buf):
                    bb = b * GB
                    pltpu.make_async_copy(
                        s_hbm.at[csrc_v.at[pl.ds(bb, GB)]],
                        rows_v.at[pl.ds(buf * GB, GB)], sems[buf]).wait()
                    rem = cnt - bb
                    ng = (jnp.minimum(rem, GB) + 15) // 16

                    def per_egroup(g, carry3):
                        eg = bb + g * 16
                        dvec = cdst_v[pl.ds(eg, 16)]
                        for l in range(16):
                            rvec = dvec.at[jnp.full((16,), l, jnp.int32)].get(
                                mode="promise_in_bounds")
                            p_row = buf * GB + g * 16 + l
                            curs = [plsc.load_gather(
                                        acc_v, [rvec, lanes + 16 * j])
                                    for j in range(D // 16)]
                            news = [jnp.maximum(
                                        curs[j],
                                        rows_v[p_row, pl.ds(16 * j, 16)])
                                    for j in range(D // 16)]
                            for j in range(D // 16):
                                plsc.store_scatter(
                                    acc_v, [rvec, lanes + 16 * j], news[j])
                        return carry3
                    lax.fori_loop(0, ng, per_egroup, 0)

                    @pl.when(b + 2 < nb)
                    def _():
                        pltpu.make_async_copy(
                            s_hbm.at[csrc_v.at[pl.ds(bb + 2 * GB, GB)]],
                            rows_v.at[pl.ds(buf * GB, GB)], sems[buf]).start()
            return carry2
        lax.fori_loop(0, (nb + 1) // 2, per_pair, 0)

    def per_chunk(ci, cnt_vec):
        base_e = ebase + ci * CHUNK
        pltpu.make_async_copy(dst_hbm.at[pl.ds(base_e, CHUNK)],
                              dstc_v, sem_d).wait()
        pltpu.make_async_copy(src_hbm.at[pl.ds(base_e, CHUNK)],
                              srcc_v, sem_s).wait()

        def per_quad(q, cnt_vec2):
            prefs, dls, svs, msks = [], [], [], []
            for u in range(4):
                g = q * 4 + u
                dv = dstc_v[pl.ds(g * 16, 16)]
                sv = srcc_v[pl.ds(g * 16, 16)]
                dl = dv - lo
                msk = (dl >= 0) & (dl < ROWS_W)
                prefs.append(plsc.cumsum(msk.astype(jnp.int32)))
                dls.append(dl)
                svs.append(sv)
                msks.append(msk)
            for u in range(4):
                pos_ix = cnt_vec2 + prefs[u] - 1
                plsc.store_scatter(cdst_v, [pos_ix], dls[u], mask=msks[u])
                plsc.store_scatter(csrc_v, [pos_ix], svs[u], mask=msks[u])
                last = prefs[u].at[jnp.full((16,), 15, jnp.int32)].get(
                    mode="promise_in_bounds")
                cnt_vec2 = cnt_vec2 + last
            return cnt_vec2
        cnt_vec = lax.fori_loop(0, CHUNK // 64, per_quad, cnt_vec)

        @pl.when(ci + 1 < ECH // CHUNK)
        def _():
            nxt = base_e + CHUNK
            pltpu.make_async_copy(dst_hbm.at[pl.ds(nxt, CHUNK)],
                                  dstc_v, sem_d).start()
            pltpu.make_async_copy(src_hbm.at[pl.ds(nxt, CHUNK)],
                                  srcc_v, sem_s).start()

        full = jnp.max(cnt_vec) > CAP - CHUNK

        @pl.when(full)
        def _():
            accumulate(cnt_vec)
        return jnp.where(full, jnp.zeros((16,), jnp.int32), cnt_vec)

    cnt_vec = lax.fori_loop(0, ECH // CHUNK, per_chunk,
                            jnp.zeros((16,), jnp.int32))
    accumulate(cnt_vec)

    pltpu.sync_copy(acc_v.at[pl.ds(0, ROWS_W)],
                    out_hbm.at[c, pl.ds(lo, ROWS_W)])


_segmax = functools.partial(
    pl.kernel,
    mesh=plsc.VectorSubcoreMesh(core_axis_name="c", subcore_axis_name="s"),
    out_type=jax.ShapeDtypeStruct((2, N_PAD, D), jnp.float32),
    scratch_types=[
        pltpu.VMEM((ROWS_W + 1, D), jnp.float32),
        pltpu.VMEM((CHUNK,), jnp.int32),
        pltpu.VMEM((CHUNK,), jnp.int32),
        pltpu.VMEM((CAP + GB + 16,), jnp.int32),
        pltpu.VMEM((CAP + GB + 16,), jnp.int32),
        pltpu.VMEM((2 * GB, D), jnp.float32),
        pltpu.SemaphoreType.DMA,
        pltpu.SemaphoreType.DMA,
        pltpu.SemaphoreType.DMA,
        pltpu.SemaphoreType.DMA,
    ],
    compiler_params=pltpu.CompilerParams(needs_layout_passes=False),
)(_segmax_body)


def kernel(x, pos, edge_index, Wh1, bh1, Wh2, bh2, Wf, bf, Wg, bg):
    src = edge_index[0]
    dst = edge_index[1]
    Wf3 = Wf[:3]
    Wfx = Wf[3:]

    grid = (N + TCB - 1) // TCB
    row_spec = pl.BlockSpec((TCB, D), lambda i: (i, 0))
    pos_spec = pl.BlockSpec((TCB, 3), lambda i: (i, 0))
    full = lambda shape: pl.BlockSpec(shape, lambda i: (0,) * len(shape))

    S, T = pl.pallas_call(
        _tables_body,
        grid=(grid,),
        in_specs=[
            row_spec, pos_spec,
            full(Wh1.shape), full((1, 64)), full(Wh2.shape), full((1, 3)),
            full(Wf3.shape), full(Wfx.shape), full((1, D)),
        ],
        out_specs=[row_spec, row_spec],
        out_shape=[jax.ShapeDtypeStruct((N, D), jnp.float32),
                   jax.ShapeDtypeStruct((N, D), jnp.float32)],
    )(x, pos, Wh1, bh1.reshape(1, 64), Wh2, bh2.reshape(1, 3),
      Wf3, Wfx, bf.reshape(1, D))

    M2 = _segmax(S, src, dst)

    out = pl.pallas_call(
        _out_body,
        grid=(grid,),
        in_specs=[row_spec, row_spec, row_spec, row_spec,
                  full(Wg.shape), full((1, D))],
        out_specs=row_spec,
        out_shape=jax.ShapeDtypeStruct((N, D), jnp.float32),
    )(x, T, M2[0], M2[1], Wg, bg.reshape(1, D))
    return out
